# baseline (speedup 1.0000x reference)
import functools

import jax
import jax.numpy as jnp
from jax import lax
from jax.experimental import pallas as pl
from jax.experimental.pallas import tpu as pltpu
from jax.experimental.pallas import tpu_sc as plsc

N = 10000
D = 128
HD = D // 2
H = 128
OUT = 64
E = 320000
L_GNN = 4

NC = 2
NS = 16
CHUNK = 128
CPT = 160
QCPT = CPT // 4
E_PAD = NS * CPT * CHUNK
NPAD = 10016
ROWS_MAIN = 624

_MESH = plsc.VectorSubcoreMesh(core_axis_name="c", subcore_axis_name="s")


@functools.partial(
    pl.kernel,
    mesh=_MESH,
    compiler_params=pltpu.CompilerParams(use_tc_tiling_on_sc=False),
    out_type=jax.ShapeDtypeStruct((NC, N, HD), jnp.float32),
    scratch_types=[
        pltpu.VMEM((QCPT, CHUNK), jnp.int32),
        pltpu.VMEM((QCPT, CHUNK), jnp.int32),
        pltpu.VMEM((CHUNK, HD), jnp.float32),
        pltpu.VMEM((CHUNK, HD), jnp.float32),
        pltpu.VMEM_SHARED((N, HD), jnp.float32),
        pltpu.VMEM_SHARED((NPAD, HD), jnp.float32),
        pltpu.SemaphoreType.DMA,
        pltpu.SemaphoreType.DMA,
        pltpu.SemaphoreType.DMA,
        pltpu.SemaphoreType.DMA,
    ],
)
def _aggregate(ha_hbm, hb_hbm, src_hbm, dst_hbm, zeros_hbm, out_hbm,
               srcv, dstv, rows0, rows1, hbuf, acc,
               gsem0, gsem1, ssem0, ssem1):
    c = lax.axis_index("c")
    s = lax.axis_index("s")
    rows = (rows0, rows1)
    gsems = (gsem0, gsem1)
    ssems = (ssem0, ssem1)
    pltpu.sync_copy(zeros_hbm, acc.at[pl.ds(s * ROWS_MAIN, ROWS_MAIN)])

    @pl.when(c == 0)
    def _load_a():
        pltpu.sync_copy(ha_hbm.at[pl.ds(s * ROWS_MAIN, ROWS_MAIN)],
                        hbuf.at[pl.ds(s * ROWS_MAIN, ROWS_MAIN)])

    @pl.when(c == 1)
    def _load_b():
        pltpu.sync_copy(hb_hbm.at[pl.ds(s * ROWS_MAIN, ROWS_MAIN)],
                        hbuf.at[pl.ds(s * ROWS_MAIN, ROWS_MAIN)])

    @pl.when(s == NS - 1)
    def _tails():
        pltpu.sync_copy(zeros_hbm.at[pl.ds(0, NPAD - NS * ROWS_MAIN)],
                        acc.at[pl.ds(NS * ROWS_MAIN, NPAD - NS * ROWS_MAIN)])

        @pl.when(c == 0)
        def _load_a_tail():
            pltpu.sync_copy(ha_hbm.at[pl.ds(NS * ROWS_MAIN, N - NS * ROWS_MAIN)],
                            hbuf.at[pl.ds(NS * ROWS_MAIN, N - NS * ROWS_MAIN)])

        @pl.when(c == 1)
        def _load_b_tail():
            pltpu.sync_copy(hb_hbm.at[pl.ds(NS * ROWS_MAIN, N - NS * ROWS_MAIN)],
                            hbuf.at[pl.ds(NS * ROWS_MAIN, N - NS * ROWS_MAIN)])

    pltpu.sync_copy(src_hbm.at[pl.ds(s * CPT, QCPT)], srcv)
    pltpu.sync_copy(dst_hbm.at[pl.ds(s * CPT, QCPT)], dstv)
    plsc.subcore_barrier()

    g_h = {0: pltpu.async_copy(hbuf.at[srcv.at[0]], rows0, gsem0)}
    s_h = {}
    for j in range(CPT):
        b = j % 2
        g_h[j].wait()
        s_h[j] = pltpu.async_copy(rows[b], acc.at[dstv.at[j % QCPT]],
                                  ssems[b], add=True)
        if j >= 1 and j % QCPT != 0:
            s_h[j - 1].wait()
        if (j + 1) % QCPT == 0 and j + 1 < CPT:
            s_h[j].wait()
            pltpu.sync_copy(src_hbm.at[pl.ds(s * CPT + j + 1, QCPT)], srcv)
            pltpu.sync_copy(dst_hbm.at[pl.ds(s * CPT + j + 1, QCPT)], dstv)
        if j + 1 < CPT:
            g_h[j + 1] = pltpu.async_copy(
                hbuf.at[srcv.at[(j + 1) % QCPT]], rows[1 - b], gsems[1 - b])
    s_h[CPT - 1].wait()

    plsc.subcore_barrier()
    pltpu.sync_copy(acc.at[pl.ds(s * ROWS_MAIN, ROWS_MAIN)],
                    out_hbm.at[c, pl.ds(s * ROWS_MAIN, ROWS_MAIN)])

    @pl.when(s == NS - 1)
    def _out_tail():
        pltpu.sync_copy(acc.at[pl.ds(NS * ROWS_MAIN, N - NS * ROWS_MAIN)],
                        out_hbm.at[c, pl.ds(NS * ROWS_MAIN, N - NS * ROWS_MAIN)])


def _mlp_body(last, scal_ref, pooled_ref, ha_ref, hb_ref, W1_ref, b1_ref,
              g1_ref, bb1_ref, W2_ref, b2_ref, g2_ref, bb2_ref, pW_ref,
              pW4_ref, pb_ref, sacc_ref, ha_out_ref, hb_out_ref, sout_ref):
    h = jnp.concatenate([ha_ref[...], hb_ref[...]], axis=1)
    pooled = jnp.concatenate([pooled_ref[0], pooled_ref[1]], axis=1)
    pooled = pooled + scal_ref[0, 0] * h
    t = jnp.dot(pooled, W1_ref[...], preferred_element_type=jnp.float32) + b1_ref[...]
    m = jnp.mean(t, axis=0, keepdims=True)
    d = t - m
    v = jnp.mean(d * d, axis=0, keepdims=True)
    t = jnp.maximum(g1_ref[...] * d * lax.rsqrt(v + 1e-5) + bb1_ref[...], 0.0)
    t = jnp.dot(t, W2_ref[...], preferred_element_type=jnp.float32) + b2_ref[...]
    m2 = jnp.mean(t, axis=0, keepdims=True)
    d2 = t - m2
    v2 = jnp.mean(d2 * d2, axis=0, keepdims=True)
    hn = jnp.maximum(g2_ref[...] * d2 * lax.rsqrt(v2 + 1e-5) + bb2_ref[...], 0.0)
    ha_out_ref[...] = hn[:, :HD]
    hb_out_ref[...] = hn[:, HD:]
    score = sacc_ref[...] + jnp.dot(
        jnp.sum(h, axis=0, keepdims=True), pW_ref[...],
        preferred_element_type=jnp.float32)
    if last:
        score = score + jnp.dot(
            jnp.sum(hn, axis=0, keepdims=True), pW4_ref[...],
            preferred_element_type=jnp.float32)
        score = score + jnp.sum(pb_ref[...], axis=0, keepdims=True)
    sout_ref[...] = score


def _mlp_call(last, scal, pooled, ha, hb, W1, b1, g1, bb1, W2, b2, g2, bb2,
              pW, pW4, pb, sacc):
    return pl.pallas_call(
        functools.partial(_mlp_body, last),
        out_shape=(jax.ShapeDtypeStruct((N, HD), jnp.float32),
                   jax.ShapeDtypeStruct((N, HD), jnp.float32),
                   jax.ShapeDtypeStruct((1, OUT), jnp.float32)),
        in_specs=[pl.BlockSpec(memory_space=pltpu.SMEM)]
        + [pl.BlockSpec(memory_space=pltpu.VMEM)] * 15,
    )(scal, pooled, ha, hb, W1, b1, g1, bb1, W2, b2, g2, bb2, pW, pW4, pb, sacc)


def kernel(x, edge_index, eps, mlp_W1, mlp_b1, mlp_bn_g, mlp_bn_b, mlp_W2,
           mlp_b2, bn_g, bn_b, pred_W, pred_b):
    ei = edge_index.astype(jnp.int32)
    dst = ei[0]
    src = ei[1]
    npe = E_PAD - E
    padv = lax.iota(jnp.int32, npe) % 16
    src_p = jnp.concatenate([src, padv]).reshape(NS * CPT, CHUNK)
    dst_p = jnp.concatenate([dst, N + padv]).reshape(NS * CPT, CHUNK)
    zeros = jnp.zeros((ROWS_MAIN, HD), jnp.float32)
    scal = (1.0 + eps).reshape(L_GNN, 1, 1)
    b1 = mlp_b1.reshape(L_GNN, 1, H)
    g1 = mlp_bn_g.reshape(L_GNN, 1, H)
    bb1 = mlp_bn_b.reshape(L_GNN, 1, H)
    b2 = mlp_b2.reshape(L_GNN, 1, H)
    g2 = bn_g.reshape(L_GNN, 1, H)
    bb2 = bn_b.reshape(L_GNN, 1, H)

    ha = x[:, :HD]
    hb = x[:, HD:]
    score = jnp.zeros((1, OUT), jnp.float32)
    for l in range(L_GNN):
        pooled = _aggregate(ha, hb, src_p, dst_p, zeros)
        ha, hb, score = _mlp_call(l == L_GNN - 1, scal[l], pooled, ha, hb,
                                  mlp_W1[l], b1[l], g1[l], bb1[l],
                                  mlp_W2[l], b2[l], g2[l], bb2[l],
                                  pred_W[l], pred_W[L_GNN], pred_b, score)
    return score


# speedup vs baseline: 1.1914x; 1.1914x over previous
import functools

import jax
import jax.numpy as jnp
from jax import lax
from jax.experimental import pallas as pl
from jax.experimental.pallas import tpu as pltpu
from jax.experimental.pallas import tpu_sc as plsc

N = 10000
D = 128
H = 128
OUT = 64
E = 320000
L_GNN = 4

NC = 2
NS = 16
NW = NC * NS
CHUNK = 128
CPW = 80
HCPW = CPW // 2
E_PAD = NW * CPW * CHUNK
NPAD = 10016
ROWS_MAIN = 624

_MESH = plsc.VectorSubcoreMesh(core_axis_name="c", subcore_axis_name="s")


@functools.partial(
    pl.kernel,
    mesh=_MESH,
    out_type=jax.ShapeDtypeStruct((NC, N, D), jnp.float32),
    scratch_types=[
        pltpu.VMEM((HCPW, CHUNK), jnp.int32),
        pltpu.VMEM((HCPW, CHUNK), jnp.int32),
        pltpu.VMEM((CHUNK, D), jnp.float32),
        pltpu.VMEM((CHUNK, D), jnp.float32),
        pltpu.VMEM_SHARED((NPAD, D), jnp.float32),
        pltpu.SemaphoreType.DMA,
        pltpu.SemaphoreType.DMA,
        pltpu.SemaphoreType.DMA,
        pltpu.SemaphoreType.DMA,
    ],
)
def _aggregate(h_hbm, src_hbm, dst_hbm, zeros_hbm, out_hbm,
               srcv, dstv, rows0, rows1, acc, gsem0, gsem1, ssem0, ssem1):
    c = lax.axis_index("c")
    s = lax.axis_index("s")
    wid = s * NC + c
    rows = (rows0, rows1)
    gsems = (gsem0, gsem1)
    ssems = (ssem0, ssem1)
    pltpu.sync_copy(zeros_hbm, acc.at[pl.ds(s * ROWS_MAIN, ROWS_MAIN)])

    @pl.when(s == NS - 1)
    def _zero_tail():
        pltpu.sync_copy(zeros_hbm.at[pl.ds(0, NPAD - NS * ROWS_MAIN)],
                        acc.at[pl.ds(NS * ROWS_MAIN, NPAD - NS * ROWS_MAIN)])

    pltpu.sync_copy(src_hbm.at[pl.ds(wid * CPW, HCPW)], srcv)
    pltpu.sync_copy(dst_hbm.at[pl.ds(wid * CPW, HCPW)], dstv)
    plsc.subcore_barrier()

    g_h = {0: pltpu.async_copy(h_hbm.at[srcv.at[0]], rows0, gsem0)}
    s_h = {}
    for j in range(CPW):
        b = j % 2
        g_h[j].wait()
        s_h[j] = pltpu.async_copy(rows[b], acc.at[dstv.at[j % HCPW]],
                                  ssems[b], add=True)
        if j >= 1 and j != HCPW:
            s_h[j - 1].wait()
        if j == HCPW - 1:
            s_h[j].wait()
            pltpu.sync_copy(src_hbm.at[pl.ds(wid * CPW + HCPW, HCPW)], srcv)
            pltpu.sync_copy(dst_hbm.at[pl.ds(wid * CPW + HCPW, HCPW)], dstv)
        if j + 1 < CPW:
            g_h[j + 1] = pltpu.async_copy(
                h_hbm.at[srcv.at[(j + 1) % HCPW]], rows[1 - b], gsems[1 - b])
    s_h[CPW - 1].wait()

    plsc.subcore_barrier()
    pltpu.sync_copy(acc.at[pl.ds(s * ROWS_MAIN, ROWS_MAIN)],
                    out_hbm.at[c, pl.ds(s * ROWS_MAIN, ROWS_MAIN)])

    @pl.when(s == NS - 1)
    def _out_tail():
        pltpu.sync_copy(acc.at[pl.ds(NS * ROWS_MAIN, N - NS * ROWS_MAIN)],
                        out_hbm.at[c, pl.ds(NS * ROWS_MAIN, N - NS * ROWS_MAIN)])


def _mlp_body(last, scal_ref, parts_ref, h_ref, W1_ref, b1_ref, g1_ref,
              bb1_ref, W2_ref, b2_ref, g2_ref, bb2_ref, pW_ref, pW4_ref,
              pb_ref, sacc_ref, hout_ref, sout_ref):
    h = h_ref[...]
    pooled = parts_ref[0] + parts_ref[1] + scal_ref[0, 0] * h
    t = jnp.dot(pooled, W1_ref[...], preferred_element_type=jnp.float32) + b1_ref[...]
    m = jnp.mean(t, axis=0, keepdims=True)
    d = t - m
    v = jnp.mean(d * d, axis=0, keepdims=True)
    t = jnp.maximum(g1_ref[...] * d * lax.rsqrt(v + 1e-5) + bb1_ref[...], 0.0)
    t = jnp.dot(t, W2_ref[...], preferred_element_type=jnp.float32) + b2_ref[...]
    m2 = jnp.mean(t, axis=0, keepdims=True)
    d2 = t - m2
    v2 = jnp.mean(d2 * d2, axis=0, keepdims=True)
    hn = jnp.maximum(g2_ref[...] * d2 * lax.rsqrt(v2 + 1e-5) + bb2_ref[...], 0.0)
    hout_ref[...] = hn
    score = sacc_ref[...] + jnp.dot(
        jnp.sum(h, axis=0, keepdims=True), pW_ref[...],
        preferred_element_type=jnp.float32)
    if last:
        score = score + jnp.dot(
            jnp.sum(hn, axis=0, keepdims=True), pW4_ref[...],
            preferred_element_type=jnp.float32)
        score = score + jnp.sum(pb_ref[...], axis=0, keepdims=True)
    sout_ref[...] = score


def _mlp_call(last, scal, parts, h, W1, b1, g1, bb1, W2, b2, g2, bb2,
              pW, pW4, pb, sacc):
    return pl.pallas_call(
        functools.partial(_mlp_body, last),
        out_shape=(jax.ShapeDtypeStruct((N, D), jnp.float32),
                   jax.ShapeDtypeStruct((1, OUT), jnp.float32)),
        in_specs=[pl.BlockSpec(memory_space=pltpu.SMEM)]
        + [pl.BlockSpec(memory_space=pltpu.VMEM)] * 14,
    )(scal, parts, h, W1, b1, g1, bb1, W2, b2, g2, bb2, pW, pW4, pb, sacc)


def kernel(x, edge_index, eps, mlp_W1, mlp_b1, mlp_bn_g, mlp_bn_b, mlp_W2,
           mlp_b2, bn_g, bn_b, pred_W, pred_b):
    ei = edge_index.astype(jnp.int32)
    dst = ei[0]
    src = ei[1]
    npe = E_PAD - E
    padv = lax.iota(jnp.int32, npe) % 16
    src_p = jnp.concatenate([src, padv]).reshape(NW * CPW, CHUNK)
    dst_p = jnp.concatenate([dst, N + padv]).reshape(NW * CPW, CHUNK)
    zeros = jnp.zeros((ROWS_MAIN, D), jnp.float32)
    scal = (1.0 + eps).reshape(L_GNN, 1, 1)
    b1 = mlp_b1.reshape(L_GNN, 1, H)
    g1 = mlp_bn_g.reshape(L_GNN, 1, H)
    bb1 = mlp_bn_b.reshape(L_GNN, 1, H)
    b2 = mlp_b2.reshape(L_GNN, 1, H)
    g2 = bn_g.reshape(L_GNN, 1, H)
    bb2 = bn_b.reshape(L_GNN, 1, H)

    h = x
    score = jnp.zeros((1, OUT), jnp.float32)
    for l in range(L_GNN):
        parts = _aggregate(h, src_p, dst_p, zeros)
        h, score = _mlp_call(l == L_GNN - 1, scal[l], parts, h,
                             mlp_W1[l], b1[l], g1[l], bb1[l],
                             mlp_W2[l], b2[l], g2[l], bb2[l],
                             pred_W[l], pred_W[L_GNN], pred_b, score)
    return score


# speedup vs baseline: 1.1933x; 1.0016x over previous
import functools

import jax
import jax.numpy as jnp
from jax import lax
from jax.experimental import pallas as pl
from jax.experimental.pallas import tpu as pltpu
from jax.experimental.pallas import tpu_sc as plsc

N = 10000
D = 128
H = 128
OUT = 64
E = 320000
L_GNN = 4

NC = 2
NS = 16
NW = NC * NS
CHUNK = 128
CPW = 80
HCPW = CPW // 2
E_PAD = NW * CPW * CHUNK
NPAD = 10016
ROWS_MAIN = 624

_MESH = plsc.VectorSubcoreMesh(core_axis_name="c", subcore_axis_name="s")


@functools.partial(
    pl.kernel,
    mesh=_MESH,
    out_type=jax.ShapeDtypeStruct((NC, N, D), jnp.float32),
    scratch_types=[
        pltpu.VMEM((HCPW, CHUNK), jnp.int32),
        pltpu.VMEM((HCPW, CHUNK), jnp.int32),
        pltpu.VMEM((CHUNK, D), jnp.float32),
        pltpu.VMEM((CHUNK, D), jnp.float32),
        pltpu.VMEM_SHARED((NPAD, D), jnp.float32),
        pltpu.SemaphoreType.DMA,
        pltpu.SemaphoreType.DMA,
        pltpu.SemaphoreType.DMA,
        pltpu.SemaphoreType.DMA,
    ],
)
def _aggregate(h_hbm, src_hbm, dst_hbm, zeros_hbm, out_hbm,
               srcv, dstv, rows0, rows1, acc, gsem0, gsem1, ssem0, ssem1):
    c = lax.axis_index("c")
    s = lax.axis_index("s")
    wid = s * NC + c
    rows = (rows0, rows1)
    gsems = (gsem0, gsem1)
    ssems = (ssem0, ssem1)
    pltpu.sync_copy(src_hbm.at[pl.ds(wid * CPW, HCPW)], srcv)
    g_h = {
        0: pltpu.async_copy(h_hbm.at[srcv.at[0]], rows0, gsem0),
        1: pltpu.async_copy(h_hbm.at[srcv.at[1]], rows1, gsem1),
    }
    pltpu.sync_copy(zeros_hbm, acc.at[pl.ds(s * ROWS_MAIN, ROWS_MAIN)])

    @pl.when(s == NS - 1)
    def _zero_tail():
        pltpu.sync_copy(zeros_hbm.at[pl.ds(0, NPAD - NS * ROWS_MAIN)],
                        acc.at[pl.ds(NS * ROWS_MAIN, NPAD - NS * ROWS_MAIN)])

    pltpu.sync_copy(dst_hbm.at[pl.ds(wid * CPW, HCPW)], dstv)
    plsc.subcore_barrier()

    s_h = {}
    for j in range(CPW):
        b = j % 2
        g_h[j].wait()
        s_h[j] = pltpu.async_copy(rows[b], acc.at[dstv.at[j % HCPW]],
                                  ssems[b], add=True)
        if j >= 1 and j != HCPW:
            s_h[j - 1].wait()
        if j == HCPW - 1:
            s_h[j].wait()
            pltpu.sync_copy(src_hbm.at[pl.ds(wid * CPW + HCPW, HCPW)], srcv)
            pltpu.sync_copy(dst_hbm.at[pl.ds(wid * CPW + HCPW, HCPW)], dstv)
        if 1 <= j < CPW - 1:
            g_h[j + 1] = pltpu.async_copy(
                h_hbm.at[srcv.at[(j + 1) % HCPW]], rows[1 - b], gsems[1 - b])
    s_h[CPW - 1].wait()

    plsc.subcore_barrier()
    pltpu.sync_copy(acc.at[pl.ds(s * ROWS_MAIN, ROWS_MAIN)],
                    out_hbm.at[c, pl.ds(s * ROWS_MAIN, ROWS_MAIN)])

    @pl.when(s == NS - 1)
    def _out_tail():
        pltpu.sync_copy(acc.at[pl.ds(NS * ROWS_MAIN, N - NS * ROWS_MAIN)],
                        out_hbm.at[c, pl.ds(NS * ROWS_MAIN, N - NS * ROWS_MAIN)])


def _mlp_body(last, scal_ref, parts_ref, h_ref, W1_ref, b1_ref, g1_ref,
              bb1_ref, W2_ref, b2_ref, g2_ref, bb2_ref, pW_ref, pW4_ref,
              pb_ref, sacc_ref, hout_ref, sout_ref):
    h = h_ref[...]
    pooled = parts_ref[0] + parts_ref[1] + scal_ref[0, 0] * h
    t = jnp.dot(pooled, W1_ref[...], preferred_element_type=jnp.float32) + b1_ref[...]
    m = jnp.mean(t, axis=0, keepdims=True)
    d = t - m
    v = jnp.mean(d * d, axis=0, keepdims=True)
    t = jnp.maximum(g1_ref[...] * d * lax.rsqrt(v + 1e-5) + bb1_ref[...], 0.0)
    t = jnp.dot(t, W2_ref[...], preferred_element_type=jnp.float32) + b2_ref[...]
    m2 = jnp.mean(t, axis=0, keepdims=True)
    d2 = t - m2
    v2 = jnp.mean(d2 * d2, axis=0, keepdims=True)
    hn = jnp.maximum(g2_ref[...] * d2 * lax.rsqrt(v2 + 1e-5) + bb2_ref[...], 0.0)
    hout_ref[...] = hn
    score = sacc_ref[...] + jnp.dot(
        jnp.sum(h, axis=0, keepdims=True), pW_ref[...],
        preferred_element_type=jnp.float32)
    if last:
        score = score + jnp.dot(
            jnp.sum(hn, axis=0, keepdims=True), pW4_ref[...],
            preferred_element_type=jnp.float32)
        score = score + jnp.sum(pb_ref[...], axis=0, keepdims=True)
    sout_ref[...] = score


def _mlp_call(last, scal, parts, h, W1, b1, g1, bb1, W2, b2, g2, bb2,
              pW, pW4, pb, sacc):
    return pl.pallas_call(
        functools.partial(_mlp_body, last),
        out_shape=(jax.ShapeDtypeStruct((N, D), jnp.float32),
                   jax.ShapeDtypeStruct((1, OUT), jnp.float32)),
        in_specs=[pl.BlockSpec(memory_space=pltpu.SMEM)]
        + [pl.BlockSpec(memory_space=pltpu.VMEM)] * 14,
    )(scal, parts, h, W1, b1, g1, bb1, W2, b2, g2, bb2, pW, pW4, pb, sacc)


def kernel(x, edge_index, eps, mlp_W1, mlp_b1, mlp_bn_g, mlp_bn_b, mlp_W2,
           mlp_b2, bn_g, bn_b, pred_W, pred_b):
    ei = edge_index.astype(jnp.int32)
    dst = ei[0]
    src = ei[1]
    npe = E_PAD - E
    padv = lax.iota(jnp.int32, npe) % 16
    src_p = jnp.concatenate([src, padv]).reshape(NW * CPW, CHUNK)
    dst_p = jnp.concatenate([dst, N + padv]).reshape(NW * CPW, CHUNK)
    zeros = jnp.zeros((ROWS_MAIN, D), jnp.float32)
    scal = (1.0 + eps).reshape(L_GNN, 1, 1)
    b1 = mlp_b1.reshape(L_GNN, 1, H)
    g1 = mlp_bn_g.reshape(L_GNN, 1, H)
    bb1 = mlp_bn_b.reshape(L_GNN, 1, H)
    b2 = mlp_b2.reshape(L_GNN, 1, H)
    g2 = bn_g.reshape(L_GNN, 1, H)
    bb2 = bn_b.reshape(L_GNN, 1, H)

    h = x
    score = jnp.zeros((1, OUT), jnp.float32)
    for l in range(L_GNN):
        parts = _aggregate(h, src_p, dst_p, zeros)
        h, score = _mlp_call(l == L_GNN - 1, scal[l], parts, h,
                             mlp_W1[l], b1[l], g1[l], bb1[l],
                             mlp_W2[l], b2[l], g2[l], bb2[l],
                             pred_W[l], pred_W[L_GNN], pred_b, score)
    return score
